# baseline (speedup 1.0000x reference)
import functools

import jax
import jax.numpy as jnp
from jax import lax
from jax.experimental import pallas as pl
from jax.experimental.pallas import tpu as pltpu
from jax.experimental.pallas import tpu_sc as plsc

HIDDEN = 128
B = 1024 * 200
NW = 32
B_PER_W = B // NW
CHUNK = 472
NBUF = 2
_SZS = [64, 408] + [CHUNK] * 12 + [160, 104]
_OFFS = [sum(_SZS[:k]) for k in range(len(_SZS))]
N_CHUNKS = len(_SZS)

_mesh = plsc.VectorSubcoreMesh(core_axis_name="c", subcore_axis_name="s")


@functools.partial(
    pl.kernel,
    out_type=jax.ShapeDtypeStruct((B, HIDDEN), jnp.float32),
    mesh=_mesh,
    scratch_types=(
        [pltpu.VMEM((B_PER_W,), jnp.int32)]
        + [pltpu.VMEM((CHUNK, HIDDEN), jnp.float32) for _ in range(NBUF)]
        + [pltpu.SemaphoreType.DMA for _ in range(2 * NBUF)]
    ),
)
def _gather_kernel(idx_hbm, w_hbm, out_hbm, idx_v, *bufs_and_sems):
    rows = bufs_and_sems[:NBUF]
    gsem = bufs_and_sems[NBUF:2 * NBUF]
    ssem = bufs_and_sems[2 * NBUF:]
    wid = lax.axis_index("s") * 2 + lax.axis_index("c")
    base = wid * B_PER_W

    pltpu.sync_copy(idx_hbm.at[pl.ds(base, _SZS[0])],
                    idx_v.at[pl.ds(0, _SZS[0])])

    def gather(i):
        return pltpu.async_copy(
            w_hbm.at[idx_v.at[pl.ds(_OFFS[i], _SZS[i])]],
            rows[i % NBUF].at[pl.ds(0, _SZS[i])],
            gsem[i % NBUF])

    def store(i):
        return pltpu.async_copy(
            rows[i % NBUF].at[pl.ds(0, _SZS[i])],
            out_hbm.at[pl.ds(base + _OFFS[i], _SZS[i])],
            ssem[i % NBUF])

    waits = [None] * N_CHUNKS
    g_handles = [gather(0)]
    pltpu.sync_copy(idx_hbm.at[pl.ds(base + _SZS[0], B_PER_W - _SZS[0])],
                    idx_v.at[pl.ds(_SZS[0], B_PER_W - _SZS[0])])
    for i in range(N_CHUNKS):
        if i + 1 < N_CHUNKS:
            j = i + 1 - NBUF
            if j >= 0:
                waits[j].wait()
                waits[j] = None
            g_handles.append(gather(i + 1))
        g_handles[i].wait()
        waits[i] = store(i)
    for w in waits:
        if w is not None:
            w.wait()


def kernel(idx, weight):
    flat = idx.reshape(-1)
    out = _gather_kernel(flat, weight)
    return out.reshape(idx.shape[0], idx.shape[1], HIDDEN)


# speedup vs baseline: 1.0211x; 1.0211x over previous
import functools

import jax
import jax.numpy as jnp
from jax import lax
from jax.experimental import pallas as pl
from jax.experimental.pallas import tpu as pltpu
from jax.experimental.pallas import tpu_sc as plsc

HIDDEN = 128
B = 1024 * 200
NW = 32
B_PER_W = B // NW
CHUNK = 472
NBUF = 2
_SZS = [CHUNK] * 13 + [B_PER_W - 13 * CHUNK]
_OFFS = [sum(_SZS[:k]) for k in range(len(_SZS))]
N_CHUNKS = len(_SZS)

_mesh = plsc.VectorSubcoreMesh(core_axis_name="c", subcore_axis_name="s")


@functools.partial(
    pl.kernel,
    out_type=jax.ShapeDtypeStruct((B, HIDDEN), jnp.float32),
    mesh=_mesh,
    scratch_types=(
        [pltpu.VMEM((B_PER_W,), jnp.int32)]
        + [pltpu.VMEM((CHUNK, HIDDEN), jnp.float32) for _ in range(NBUF)]
        + [pltpu.SemaphoreType.DMA for _ in range(2 * NBUF)]
    ),
)
def _gather_kernel(idx_hbm, w_hbm, out_hbm, idx_v, *bufs_and_sems):
    rows = bufs_and_sems[:NBUF]
    gsem = bufs_and_sems[NBUF:2 * NBUF]
    ssem = bufs_and_sems[2 * NBUF:]
    wid = lax.axis_index("s") * 2 + lax.axis_index("c")
    base = wid * B_PER_W

    pltpu.sync_copy(idx_hbm.at[pl.ds(base, _SZS[0])],
                    idx_v.at[pl.ds(0, _SZS[0])])

    def gather(i):
        return pltpu.async_copy(
            w_hbm.at[idx_v.at[pl.ds(_OFFS[i], _SZS[i])]],
            rows[i % NBUF].at[pl.ds(0, _SZS[i])],
            gsem[i % NBUF])

    def store(i):
        return pltpu.async_copy(
            rows[i % NBUF].at[pl.ds(0, _SZS[i])],
            out_hbm.at[pl.ds(base + _OFFS[i], _SZS[i])],
            ssem[i % NBUF])

    waits = [None] * N_CHUNKS
    g_handles = [gather(0)]
    pltpu.sync_copy(idx_hbm.at[pl.ds(base + _SZS[0], B_PER_W - _SZS[0])],
                    idx_v.at[pl.ds(_SZS[0], B_PER_W - _SZS[0])])
    for i in range(N_CHUNKS):
        if i + 1 < N_CHUNKS:
            j = i + 1 - NBUF
            if j >= 0:
                waits[j].wait()
                waits[j] = None
            g_handles.append(gather(i + 1))
        g_handles[i].wait()
        waits[i] = store(i)
    for w in waits:
        if w is not None:
            w.wait()


def kernel(idx, weight):
    flat = idx.reshape(-1)
    out = _gather_kernel(flat, weight)
    return out.reshape(idx.shape[0], idx.shape[1], HIDDEN)


# speedup vs baseline: 1.0238x; 1.0027x over previous
import functools

import jax
import jax.numpy as jnp
from jax import lax
from jax.experimental import pallas as pl
from jax.experimental.pallas import tpu as pltpu
from jax.experimental.pallas import tpu_sc as plsc

HIDDEN = 128
B = 1024 * 200
NW = 32
B_PER_W = B // NW
CHUNK = 472
NBUF = 2
_SZS = [CHUNK] * 13 + [B_PER_W - 13 * CHUNK]
_OFFS = [sum(_SZS[:k]) for k in range(len(_SZS))]
N_CHUNKS = len(_SZS)

_mesh = plsc.VectorSubcoreMesh(core_axis_name="c", subcore_axis_name="s")


@functools.partial(
    pl.kernel,
    out_type=jax.ShapeDtypeStruct((B, HIDDEN), jnp.float32),
    mesh=_mesh,
    scratch_types=(
        [pltpu.VMEM((B_PER_W,), jnp.int32)]
        + [pltpu.VMEM((CHUNK, HIDDEN), jnp.float32) for _ in range(NBUF)]
        + [pltpu.SemaphoreType.DMA for _ in range(2 * NBUF)]
    ),
)
def _gather_kernel(idx_hbm, w_hbm, out_hbm, idx_v, *bufs_and_sems):
    rows = bufs_and_sems[:NBUF]
    gsem = bufs_and_sems[NBUF:2 * NBUF]
    ssem = bufs_and_sems[2 * NBUF:]
    wid = lax.axis_index("c") * 16 + lax.axis_index("s")
    base = wid * B_PER_W

    pltpu.sync_copy(idx_hbm.at[pl.ds(base, _SZS[0])],
                    idx_v.at[pl.ds(0, _SZS[0])])

    def gather(i):
        return pltpu.async_copy(
            w_hbm.at[idx_v.at[pl.ds(_OFFS[i], _SZS[i])]],
            rows[i % NBUF].at[pl.ds(0, _SZS[i])],
            gsem[i % NBUF])

    def store(i):
        return pltpu.async_copy(
            rows[i % NBUF].at[pl.ds(0, _SZS[i])],
            out_hbm.at[pl.ds(base + _OFFS[i], _SZS[i])],
            ssem[i % NBUF])

    waits = [None] * N_CHUNKS
    g_handles = [gather(0)]
    pltpu.sync_copy(idx_hbm.at[pl.ds(base + _SZS[0], B_PER_W - _SZS[0])],
                    idx_v.at[pl.ds(_SZS[0], B_PER_W - _SZS[0])])
    for i in range(N_CHUNKS):
        if i + 1 < N_CHUNKS:
            j = i + 1 - NBUF
            if j >= 0:
                waits[j].wait()
                waits[j] = None
            g_handles.append(gather(i + 1))
        g_handles[i].wait()
        waits[i] = store(i)
    for w in waits:
        if w is not None:
            w.wait()


def kernel(idx, weight):
    flat = idx.reshape(-1)
    out = _gather_kernel(flat, weight)
    return out.reshape(idx.shape[0], idx.shape[1], HIDDEN)
